# Initial kernel scaffold; baseline (speedup 1.0000x reference)
import functools

import jax
import jax.numpy as jnp
from jax import lax
from jax.experimental import pallas as pl
from jax.experimental.pallas import tpu as pltpu
from jax.experimental.pallas import tpu_sc as plsc

_K = 8192
_D = 32
_BETA = 0.25
_BN = 256


def _dist_argmin_body(nb, z_ref, cbt_ref, idx_ref, loss_ref):
    i = pl.program_id(0)
    z = z_ref[...]
    cbt = cbt_ref[...]
    znorm = jnp.sum(z * z, axis=1, keepdims=True)
    cnorm = jnp.sum(cbt * cbt, axis=0, keepdims=True)
    mm = jnp.dot(z, cbt, preferred_element_type=jnp.float32)
    dist = znorm + cnorm - 2.0 * mm
    mind = jnp.min(dist, axis=1, keepdims=True)
    kiota = lax.broadcasted_iota(jnp.int32, dist.shape, 1)
    idx = jnp.min(jnp.where(dist <= mind, kiota, _K), axis=1)
    idx_ref[0, 0, :] = idx.astype(jnp.int32)

    @pl.when(i == 0)
    def _init():
        loss_ref[0, 0] = 0.0

    loss_ref[0, 0] += jnp.sum(mind)

    @pl.when(i == nb - 1)
    def _finish():
        n_total = nb * _BN
        loss_ref[0, 0] = loss_ref[0, 0] * ((1.0 + _BETA) / (n_total * _D))


def _dist_argmin(flat, cbt):
    n = flat.shape[0]
    nb = n // _BN
    return pl.pallas_call(
        functools.partial(_dist_argmin_body, nb),
        grid=(nb,),
        in_specs=[
            pl.BlockSpec((_BN, _D), lambda i: (i, 0)),
            pl.BlockSpec((_D, _K), lambda i: (0, 0)),
        ],
        out_specs=[
            pl.BlockSpec((1, 1, _BN), lambda i: (i, 0, 0)),
            pl.BlockSpec((1, 1), lambda i: (0, 0)),
        ],
        out_shape=[
            jax.ShapeDtypeStruct((nb, 1, _BN), jnp.int32),
            jax.ShapeDtypeStruct((1, 1), jnp.float32),
        ],
    )(flat, cbt)


_SC_INFO = plsc.get_sparse_core_info()
_NW = _SC_INFO.num_cores * _SC_INFO.num_subcores
_CH = 128


def _make_sc_gather(n):
    nrows = n // _CH
    rows_w = nrows // _NW
    mesh = plsc.VectorSubcoreMesh(core_axis_name="c", subcore_axis_name="s")

    @functools.partial(
        pl.kernel,
        mesh=mesh,
        out_type=jax.ShapeDtypeStruct((nrows, _CH, _D), jnp.float32),
        scratch_types=[
            pltpu.VMEM((rows_w, _CH), jnp.int32),
            pltpu.VMEM((rows_w, _CH, _D), jnp.float32),
            pltpu.SemaphoreType.DMA,
        ],
    )
    def gather(table_hbm, idx_hbm, out_hbm, idx_v, rows_v, sem):
        wid = lax.axis_index("s") * _SC_INFO.num_cores + lax.axis_index("c")
        base = wid * rows_w
        pltpu.sync_copy(idx_hbm.at[pl.ds(base, rows_w)], idx_v)
        copies = [
            pltpu.async_copy(table_hbm.at[idx_v.at[j]], rows_v.at[j], sem)
            for j in range(rows_w)
        ]
        for c in copies:
            c.wait()
        pltpu.sync_copy(rows_v, out_hbm.at[pl.ds(base, rows_w)])

    return gather


def kernel(z_content, codebook):
    z = z_content
    flat = z.reshape(-1, _D)
    n = flat.shape[0]
    cbt = codebook.T

    idx3, loss2 = _dist_argmin(flat, cbt)
    idx = idx3.reshape(n)
    loss = loss2[0, 0]

    zq = _make_sc_gather(n)(codebook, idx.reshape(n // _CH, _CH))
    fe_output = zq.reshape(-1, 1)
    return fe_output, loss, idx


# speedup vs baseline: 1.3279x; 1.3279x over previous
import functools

import jax
import jax.numpy as jnp
from jax import lax
from jax.experimental import pallas as pl
from jax.experimental.pallas import tpu as pltpu
from jax.experimental.pallas import tpu_sc as plsc

_K = 8192
_D = 32
_BETA = 0.25
_BN = 256


def _dist_argmin_body(nb, z_ref, cb_ref, cbt_ref, idx_ref, loss_ref):
    i = pl.program_id(0)
    z = z_ref[...]
    cb = cb_ref[...]
    cbt = cbt_ref[...]
    znorm = jnp.sum(z * z, axis=1, keepdims=True)
    cnorm = jnp.sum(cbt * cbt, axis=0, keepdims=True)
    s = (2.0 * z).astype(jnp.bfloat16)
    cbb = cb.astype(jnp.bfloat16)
    mm = lax.dot_general(s, cbb, (((1,), (1,)), ((), ())),
                         preferred_element_type=jnp.float32)
    dist = (znorm + cnorm) - mm
    h = _K // 2
    d1 = dist[:, :h]
    d2 = dist[:, h:]
    m1 = jnp.min(d1, axis=1, keepdims=True)
    m2 = jnp.min(d2, axis=1, keepdims=True)
    kiota = lax.broadcasted_iota(jnp.int32, d1.shape, 1)
    i1 = jnp.min(jnp.where(d1 <= m1, kiota, h), axis=1)
    i2 = jnp.min(jnp.where(d2 <= m2, kiota, h), axis=1) + h
    m1b = m1.astype(jnp.bfloat16).astype(jnp.float32)
    take2 = m2 < m1b
    idx = jnp.where(take2[:, 0], i2, i1)
    mind = jnp.where(take2, m2, m1)
    idx_ref[0, 0, :] = idx.astype(jnp.int32)

    @pl.when(i == 0)
    def _init():
        loss_ref[...] = jnp.zeros((1, 1), jnp.float32)

    loss_ref[...] += jnp.sum(mind).reshape(1, 1)

    @pl.when(i == nb - 1)
    def _finish():
        n_total = nb * _BN
        loss_ref[...] = loss_ref[...] * ((1.0 + _BETA) / (n_total * _D))


def _dist_argmin(flat, cb, cbt):
    n = flat.shape[0]
    nb = n // _BN
    return pl.pallas_call(
        functools.partial(_dist_argmin_body, nb),
        grid=(nb,),
        in_specs=[
            pl.BlockSpec((_BN, _D), lambda i: (i, 0)),
            pl.BlockSpec((_K, _D), lambda i: (0, 0)),
            pl.BlockSpec((_D, _K), lambda i: (0, 0)),
        ],
        out_specs=[
            pl.BlockSpec((1, 1, _BN), lambda i: (i, 0, 0)),
            pl.BlockSpec((1, 1), lambda i: (0, 0)),
        ],
        out_shape=[
            jax.ShapeDtypeStruct((nb, 1, _BN), jnp.int32),
            jax.ShapeDtypeStruct((1, 1), jnp.float32),
        ],
    )(flat, cb, cbt)


_CH = 128


def _make_sc_gather(n):
    info = plsc.get_sparse_core_info()
    num_cores = info.num_cores
    nw = num_cores * info.num_subcores
    nrows = n // _CH
    rows_w = nrows // nw
    mesh = plsc.VectorSubcoreMesh(core_axis_name="c", subcore_axis_name="s")

    @functools.partial(
        pl.kernel,
        mesh=mesh,
        compiler_params=pltpu.CompilerParams(use_tc_tiling_on_sc=False),
        out_type=jax.ShapeDtypeStruct((nrows, _CH, _D), jnp.float32),
        scratch_types=[
            pltpu.VMEM((rows_w, _CH), jnp.int32),
            pltpu.VMEM((rows_w, _CH, _D), jnp.float32),
            pltpu.SemaphoreType.DMA,
        ],
    )
    def gather(table_hbm, idx_hbm, out_hbm, idx_v, rows_v, sem):
        wid = lax.axis_index("s") * num_cores + lax.axis_index("c")
        base = wid * rows_w
        pltpu.sync_copy(idx_hbm.at[pl.ds(base, rows_w)], idx_v)
        copies = [
            pltpu.async_copy(table_hbm.at[idx_v.at[j]], rows_v.at[j], sem)
            for j in range(rows_w)
        ]
        for c in copies:
            c.wait()
        pltpu.sync_copy(rows_v, out_hbm.at[pl.ds(base, rows_w)])

    return gather


def kernel(z_content, codebook):
    z = z_content
    flat = z.reshape(-1, _D)
    n = flat.shape[0]
    cbt = codebook.T

    idx3, loss2 = _dist_argmin(flat, codebook, cbt)
    idx = idx3.reshape(n)
    loss = loss2[0, 0]

    zq = _make_sc_gather(n)(codebook, idx.reshape(n // _CH, _CH))
    fe_output = zq.reshape(-1, 1)
    return fe_output, loss, idx


# speedup vs baseline: 1.8229x; 1.3728x over previous
import functools

import jax
import jax.numpy as jnp
from jax import lax
from jax.experimental import pallas as pl
from jax.experimental.pallas import tpu as pltpu
from jax.experimental.pallas import tpu_sc as plsc

_K = 8192
_D = 32
_BETA = 0.25
_BN = 512


def _dist_argmin_body(nb, z_ref, cb_ref, cbt_ref, idx_ref, loss_ref):
    i = pl.program_id(0)
    z = z_ref[...]
    cb = cb_ref[...]
    cbt = cbt_ref[...]
    znorm = jnp.sum(z * z, axis=1, keepdims=True)
    cnorm = jnp.sum(cbt * cbt, axis=0, keepdims=True)
    s = (2.0 * z).astype(jnp.bfloat16)
    cbb = cb.astype(jnp.bfloat16)
    mm = lax.dot_general(s, cbb, (((1,), (1,)), ((), ())),
                         preferred_element_type=jnp.float32)
    dist = (znorm + cnorm) - mm
    h = _K // 2
    rc = 64
    lane = lax.broadcasted_iota(jnp.int32, (rc, 128), 1)
    idx_list, mind_list = [], []
    for r0 in range(0, _BN, rc):
        res = []
        for base in (0, h):
            cv = dist[r0:r0 + rc, base:base + 128]
            cg = jnp.zeros((rc, 128), jnp.int32)
            for g in range(1, h // 128):
                blk = dist[r0:r0 + rc, base + g * 128:base + (g + 1) * 128]
                lt = blk < cv
                cv = jnp.where(lt, blk, cv)
                cg = jnp.where(lt, g, cg)
            m = jnp.min(cv, axis=1, keepdims=True)
            kv = cg * 128 + lane + base
            ki = jnp.min(jnp.where(cv <= m, kv, _K), axis=1, keepdims=True)
            res.append((m, ki))
        (m1, i1), (m2, i2) = res
        m1b = m1.astype(jnp.bfloat16).astype(jnp.float32)
        take2 = m2 < m1b
        idx_list.append(jnp.where(take2, i2, i1))
        mind_list.append(jnp.where(take2, m2, m1))
    idx = jnp.concatenate(idx_list, axis=0)
    mind = jnp.concatenate(mind_list, axis=0)
    idx_ref[0, 0, :] = idx[:, 0].astype(jnp.int32)

    @pl.when(i == 0)
    def _init():
        loss_ref[...] = jnp.zeros((1, 1), jnp.float32)

    loss_ref[...] += jnp.sum(mind).reshape(1, 1)

    @pl.when(i == nb - 1)
    def _finish():
        n_total = nb * _BN
        loss_ref[...] = loss_ref[...] * ((1.0 + _BETA) / (n_total * _D))


def _dist_argmin(flat, cb, cbt):
    n = flat.shape[0]
    nb = n // _BN
    return pl.pallas_call(
        functools.partial(_dist_argmin_body, nb),
        grid=(nb,),
        in_specs=[
            pl.BlockSpec((_BN, _D), lambda i: (i, 0)),
            pl.BlockSpec((_K, _D), lambda i: (0, 0)),
            pl.BlockSpec((_D, _K), lambda i: (0, 0)),
        ],
        out_specs=[
            pl.BlockSpec((1, 1, _BN), lambda i: (i, 0, 0)),
            pl.BlockSpec((1, 1), lambda i: (0, 0)),
        ],
        out_shape=[
            jax.ShapeDtypeStruct((nb, 1, _BN), jnp.int32),
            jax.ShapeDtypeStruct((1, 1), jnp.float32),
        ],
    )(flat, cb, cbt)


_CH = 128


def _make_sc_gather(n):
    info = plsc.get_sparse_core_info()
    num_cores = info.num_cores
    nw = num_cores * info.num_subcores
    nrows = n // _CH
    rows_w = nrows // nw
    mesh = plsc.VectorSubcoreMesh(core_axis_name="c", subcore_axis_name="s")

    @functools.partial(
        pl.kernel,
        mesh=mesh,
        compiler_params=pltpu.CompilerParams(use_tc_tiling_on_sc=False),
        out_type=jax.ShapeDtypeStruct((nrows, _CH, _D), jnp.float32),
        scratch_types=[
            pltpu.VMEM((rows_w, _CH), jnp.int32),
            pltpu.VMEM((rows_w, _CH, _D), jnp.float32),
            pltpu.SemaphoreType.DMA,
        ],
    )
    def gather(table_hbm, idx_hbm, out_hbm, idx_v, rows_v, sem):
        wid = lax.axis_index("s") * num_cores + lax.axis_index("c")
        base = wid * rows_w
        pltpu.sync_copy(idx_hbm.at[pl.ds(base, rows_w)], idx_v)
        copies = [
            pltpu.async_copy(table_hbm.at[idx_v.at[j]], rows_v.at[j], sem)
            for j in range(rows_w)
        ]
        for c in copies:
            c.wait()
        pltpu.sync_copy(rows_v, out_hbm.at[pl.ds(base, rows_w)])

    return gather


def kernel(z_content, codebook):
    z = z_content
    flat = z.reshape(-1, _D)
    n = flat.shape[0]
    cbt = codebook.T

    idx3, loss2 = _dist_argmin(flat, codebook, cbt)
    idx = idx3.reshape(n)
    loss = loss2[0, 0]

    zq = _make_sc_gather(n)(codebook, idx.reshape(n // _CH, _CH))
    fe_output = zq.reshape(-1, 1)
    return fe_output, loss, idx


# speedup vs baseline: 1.8247x; 1.0010x over previous
import functools

import jax
import jax.numpy as jnp
from jax import lax
from jax.experimental import pallas as pl
from jax.experimental.pallas import tpu as pltpu
from jax.experimental.pallas import tpu_sc as plsc

_K = 8192
_D = 32
_BETA = 0.25
_BN = 512


def _dist_argmin_body(nb, z_ref, cb_ref, cbt_ref, idx_ref, loss_ref):
    i = pl.program_id(0)
    z = z_ref[...]
    cb = cb_ref[...]
    cbt = cbt_ref[...]
    znorm = jnp.sum(z * z, axis=1, keepdims=True)
    cnorm = jnp.sum(cbt * cbt, axis=0, keepdims=True)
    s = (2.0 * z).astype(jnp.bfloat16)
    cbb = cb.astype(jnp.bfloat16)
    mm = lax.dot_general(s, cbb, (((1,), (1,)), ((), ())),
                         preferred_element_type=jnp.float32)
    h = _K // 2
    rc = 64
    lane = lax.broadcasted_iota(jnp.int32, (rc, 128), 1)
    idx_list, mind_list = [], []
    for r0 in range(0, _BN, rc):
        zn_rc = znorm[r0:r0 + rc, :]
        res = []
        for base in (0, h):
            cv = (zn_rc + cnorm[:, base:base + 128]) - mm[r0:r0 + rc, base:base + 128]
            cg = jnp.zeros((rc, 128), jnp.int32)
            for g in range(1, h // 128):
                c0 = base + g * 128
                blk = (zn_rc + cnorm[:, c0:c0 + 128]) - mm[r0:r0 + rc, c0:c0 + 128]
                lt = blk < cv
                cv = jnp.where(lt, blk, cv)
                cg = jnp.where(lt, g, cg)
            m = jnp.min(cv, axis=1, keepdims=True)
            kv = cg * 128 + lane + base
            ki = jnp.min(jnp.where(cv <= m, kv, _K), axis=1, keepdims=True)
            res.append((m, ki))
        (m1, i1), (m2, i2) = res
        m1b = m1.astype(jnp.bfloat16).astype(jnp.float32)
        take2 = m2 < m1b
        idx_list.append(jnp.where(take2, i2, i1))
        mind_list.append(jnp.where(take2, m2, m1))
    idx = jnp.concatenate(idx_list, axis=0)
    mind = jnp.concatenate(mind_list, axis=0)
    idx_ref[0, 0, :] = idx[:, 0].astype(jnp.int32)

    @pl.when(i == 0)
    def _init():
        loss_ref[...] = jnp.zeros((1, 1), jnp.float32)

    loss_ref[...] += jnp.sum(mind).reshape(1, 1)

    @pl.when(i == nb - 1)
    def _finish():
        n_total = nb * _BN
        loss_ref[...] = loss_ref[...] * ((1.0 + _BETA) / (n_total * _D))


def _dist_argmin(flat, cb, cbt):
    n = flat.shape[0]
    nb = n // _BN
    return pl.pallas_call(
        functools.partial(_dist_argmin_body, nb),
        grid=(nb,),
        in_specs=[
            pl.BlockSpec((_BN, _D), lambda i: (i, 0)),
            pl.BlockSpec((_K, _D), lambda i: (0, 0)),
            pl.BlockSpec((_D, _K), lambda i: (0, 0)),
        ],
        out_specs=[
            pl.BlockSpec((1, 1, _BN), lambda i: (i, 0, 0)),
            pl.BlockSpec((1, 1), lambda i: (0, 0)),
        ],
        out_shape=[
            jax.ShapeDtypeStruct((nb, 1, _BN), jnp.int32),
            jax.ShapeDtypeStruct((1, 1), jnp.float32),
        ],
    )(flat, cb, cbt)


_CH = 128


def _make_sc_gather(n):
    info = plsc.get_sparse_core_info()
    num_cores = info.num_cores
    nw = num_cores * info.num_subcores
    nrows = n // _CH
    rows_w = nrows // nw
    mesh = plsc.VectorSubcoreMesh(core_axis_name="c", subcore_axis_name="s")

    @functools.partial(
        pl.kernel,
        mesh=mesh,
        compiler_params=pltpu.CompilerParams(use_tc_tiling_on_sc=False),
        out_type=jax.ShapeDtypeStruct((nrows, _CH, _D), jnp.float32),
        scratch_types=[
            pltpu.VMEM((rows_w, _CH), jnp.int32),
            pltpu.VMEM((rows_w, _CH, _D), jnp.float32),
            pltpu.SemaphoreType.DMA,
        ],
    )
    def gather(table_hbm, idx_hbm, out_hbm, idx_v, rows_v, sem):
        wid = lax.axis_index("s") * num_cores + lax.axis_index("c")
        base = wid * rows_w
        pltpu.sync_copy(idx_hbm.at[pl.ds(base, rows_w)], idx_v)
        copies = [
            pltpu.async_copy(table_hbm.at[idx_v.at[j]], rows_v.at[j], sem)
            for j in range(rows_w)
        ]
        for c in copies:
            c.wait()
        pltpu.sync_copy(rows_v, out_hbm.at[pl.ds(base, rows_w)])

    return gather


def kernel(z_content, codebook):
    z = z_content
    flat = z.reshape(-1, _D)
    n = flat.shape[0]
    cbt = codebook.T

    idx3, loss2 = _dist_argmin(flat, codebook, cbt)
    idx = idx3.reshape(n)
    loss = loss2[0, 0]

    zq = _make_sc_gather(n)(codebook, idx.reshape(n // _CH, _CH))
    fe_output = zq.reshape(-1, 1)
    return fe_output, loss, idx


# speedup vs baseline: 1.8965x; 1.0394x over previous
import functools

import jax
import jax.numpy as jnp
from jax import lax
from jax.experimental import pallas as pl
from jax.experimental.pallas import tpu as pltpu
from jax.experimental.pallas import tpu_sc as plsc

_K = 8192
_D = 32
_BETA = 0.25
_BN = 1024


def _dist_argmin_body(nb, z_ref, cb_ref, cbt_ref, idx_ref, loss_ref):
    i = pl.program_id(0)
    z = z_ref[...]
    cb = cb_ref[...]
    cbt = cbt_ref[...]
    znorm = jnp.sum(z * z, axis=1, keepdims=True)
    cnorm = jnp.sum(cbt * cbt, axis=0, keepdims=True)
    s = (2.0 * z).astype(jnp.bfloat16)
    cbb = cb.astype(jnp.bfloat16)
    mm = lax.dot_general(s, cbb, (((1,), (1,)), ((), ())),
                         preferred_element_type=jnp.float32)
    h = _K // 2
    rc = 64
    lane = lax.broadcasted_iota(jnp.int32, (rc, 128), 1)
    idx_list, mind_list = [], []
    for r0 in range(0, _BN, rc):
        zn_rc = znorm[r0:r0 + rc, :]
        res = []
        for base in (0, h):
            cv = (zn_rc + cnorm[:, base:base + 128]) - mm[r0:r0 + rc, base:base + 128]
            cg = jnp.zeros((rc, 128), jnp.int32)
            for g in range(1, h // 128):
                c0 = base + g * 128
                blk = (zn_rc + cnorm[:, c0:c0 + 128]) - mm[r0:r0 + rc, c0:c0 + 128]
                lt = blk < cv
                cv = jnp.where(lt, blk, cv)
                cg = jnp.where(lt, g, cg)
            m = jnp.min(cv, axis=1, keepdims=True)
            kv = cg * 128 + lane + base
            ki = jnp.min(jnp.where(cv <= m, kv, _K), axis=1, keepdims=True)
            res.append((m, ki))
        (m1, i1), (m2, i2) = res
        m1b = m1.astype(jnp.bfloat16).astype(jnp.float32)
        take2 = m2 < m1b
        idx_list.append(jnp.where(take2, i2, i1))
        mind_list.append(jnp.where(take2, m2, m1))
    idx = jnp.concatenate(idx_list, axis=0)
    mind = jnp.concatenate(mind_list, axis=0)
    idx_ref[0, 0, :] = idx[:, 0].astype(jnp.int32)

    @pl.when(i == 0)
    def _init():
        loss_ref[...] = jnp.zeros((1, 1), jnp.float32)

    loss_ref[...] += jnp.sum(mind).reshape(1, 1)

    @pl.when(i == nb - 1)
    def _finish():
        n_total = nb * _BN
        loss_ref[...] = loss_ref[...] * ((1.0 + _BETA) / (n_total * _D))


def _dist_argmin(flat, cb, cbt):
    n = flat.shape[0]
    nb = n // _BN
    return pl.pallas_call(
        functools.partial(_dist_argmin_body, nb),
        grid=(nb,),
        in_specs=[
            pl.BlockSpec((_BN, _D), lambda i: (i, 0)),
            pl.BlockSpec((_K, _D), lambda i: (0, 0)),
            pl.BlockSpec((_D, _K), lambda i: (0, 0)),
        ],
        out_specs=[
            pl.BlockSpec((1, 1, _BN), lambda i: (i, 0, 0)),
            pl.BlockSpec((1, 1), lambda i: (0, 0)),
        ],
        out_shape=[
            jax.ShapeDtypeStruct((nb, 1, _BN), jnp.int32),
            jax.ShapeDtypeStruct((1, 1), jnp.float32),
        ],
    )(flat, cb, cbt)


_CH = 128


def _make_sc_gather(n):
    info = plsc.get_sparse_core_info()
    num_cores = info.num_cores
    nw = num_cores * info.num_subcores
    nrows = n // _CH
    rows_w = nrows // nw
    mesh = plsc.VectorSubcoreMesh(core_axis_name="c", subcore_axis_name="s")

    @functools.partial(
        pl.kernel,
        mesh=mesh,
        compiler_params=pltpu.CompilerParams(use_tc_tiling_on_sc=False),
        out_type=jax.ShapeDtypeStruct((nrows, _CH, _D), jnp.float32),
        scratch_types=[
            pltpu.VMEM((rows_w, _CH), jnp.int32),
            pltpu.VMEM((rows_w, _CH, _D), jnp.float32),
            pltpu.SemaphoreType.DMA,
        ],
    )
    def gather(table_hbm, idx_hbm, out_hbm, idx_v, rows_v, sem):
        wid = lax.axis_index("s") * num_cores + lax.axis_index("c")
        base = wid * rows_w
        pltpu.sync_copy(idx_hbm.at[pl.ds(base, rows_w)], idx_v)
        copies = [
            pltpu.async_copy(table_hbm.at[idx_v.at[j]], rows_v.at[j], sem)
            for j in range(rows_w)
        ]
        for c in copies:
            c.wait()
        pltpu.sync_copy(rows_v, out_hbm.at[pl.ds(base, rows_w)])

    return gather


def kernel(z_content, codebook):
    z = z_content
    flat = z.reshape(-1, _D)
    n = flat.shape[0]
    cbt = codebook.T

    idx3, loss2 = _dist_argmin(flat, codebook, cbt)
    idx = idx3.reshape(n)
    loss = loss2[0, 0]

    zq = _make_sc_gather(n)(codebook, idx.reshape(n // _CH, _CH))
    fe_output = zq.reshape(-1, 1)
    return fe_output, loss, idx
